# baseline (speedup 1.0000x reference)
import jax
import jax.numpy as jnp
from jax.experimental import pallas as pl
from jax.experimental.pallas import tpu as pltpu

_BT = 1024


def _router_block(x_ref, w_ref, b_ref, o_ref, buf0, buf1, sem):
    i = pl.program_id(0)
    nstep = pl.num_programs(0)
    bufs = (buf0, buf1)

    def fetch(block, slot):
        pltpu.make_async_copy(
            x_ref.at[pl.ds(block * _BT, _BT), :], bufs[slot], sem.at[slot],
        ).start()

    @pl.when(i == 0)
    def _prologue():
        fetch(0, 0)

    def run(slot):
        @pl.when(nstep > i + 1)
        def _prefetch():
            fetch(i + 1, 1 - slot)

        pltpu.make_async_copy(
            x_ref.at[pl.ds(i * _BT, _BT), :], bufs[slot], sem.at[slot],
        ).wait()
        xb = bufs[slot][...].astype(jnp.bfloat16)
        wb = w_ref[...].astype(jnp.bfloat16)
        acc = jax.lax.dot_general(
            wb, xb, (((1,), (1,)), ((), ())),
            preferred_element_type=jnp.float32)
        o_ref[...] = acc + jnp.transpose(b_ref[...], (1, 0))

    parity = jax.lax.rem(i, 2)

    @pl.when(parity == 0)
    def _even():
        run(0)

    @pl.when(parity == 1)
    def _odd():
        run(1)


def kernel(x, W, b):
    tokens, d = x.shape
    e = W.shape[0]
    b2 = b.reshape(1, e)
    logits_t = pl.pallas_call(
        _router_block,
        grid=(tokens // _BT,),
        in_specs=[
            pl.BlockSpec(memory_space=pltpu.MemorySpace.HBM),
            pl.BlockSpec((e, d), lambda i: (0, 0)),
            pl.BlockSpec((1, e), lambda i: (0, 0)),
        ],
        out_specs=pl.BlockSpec((e, _BT), lambda i: (0, i)),
        out_shape=jax.ShapeDtypeStruct((e, tokens), jnp.float32),
        scratch_shapes=[
            pltpu.VMEM((_BT, d), jnp.float32),
            pltpu.VMEM((_BT, d), jnp.float32),
            pltpu.SemaphoreType.DMA((2,)),
        ],
    )(x, W, b2)
    return logits_t.T


# speedup vs baseline: 1.0008x; 1.0008x over previous
import jax
import jax.numpy as jnp
from jax.experimental import pallas as pl
from jax.experimental.pallas import tpu as pltpu

_BT = 1024
_NQ = 4
_QT = _BT // _NQ


def _router_block(x_ref, w_ref, b_ref, o_ref, buf0, buf1, sem):
    i = pl.program_id(0)
    nstep = pl.num_programs(0)
    bufs = (buf0, buf1)
    wb = w_ref[...].astype(jnp.bfloat16)
    bias = jnp.transpose(b_ref[...], (1, 0))

    def full_copy(block, slot):
        return pltpu.make_async_copy(
            x_ref.at[pl.ds(block * _BT, _BT), :], bufs[slot], sem.at[slot])

    def quarter_copy(block, slot, q):
        return pltpu.make_async_copy(
            x_ref.at[pl.ds(block * _BT + q * _QT, _QT), :],
            bufs[slot].at[pl.ds(q * _QT, _QT), :], sem.at[2 + q])

    def emit(acc, sl):
        o_ref[:, sl] = acc + bias

    def dot(xv):
        return jax.lax.dot_general(
            wb, xv, (((1,), (1,)), ((), ())),
            preferred_element_type=jnp.float32)

    def compute_quarters(block, slot):
        for q in range(_NQ):
            quarter_copy(block, slot, q).wait()
            xq = bufs[slot][q * _QT:(q + 1) * _QT, :].astype(jnp.bfloat16)
            emit(dot(xq), pl.ds(q * _QT, _QT))

    @pl.when(i == 0)
    def _head():
        for q in range(_NQ):
            quarter_copy(0, 0, q).start()
        full_copy(1, 1).start()
        compute_quarters(0, 0)

    def run(slot):
        nxt = i + 1

        @pl.when(nxt < nstep - 1)
        def _pf_full():
            full_copy(nxt, 1 - slot).start()

        @pl.when(nxt == nstep - 1)
        def _pf_quarters():
            for q in range(_NQ):
                quarter_copy(nxt, 1 - slot, q).start()

        @pl.when(i < nstep - 1)
        def _steady():
            full_copy(i, slot).wait()
            xb = bufs[slot][...].astype(jnp.bfloat16)
            emit(dot(xb), slice(None))

        @pl.when(i == nstep - 1)
        def _tail():
            compute_quarters(i, slot)

    parity = jax.lax.rem(i, 2)

    @pl.when((i > 0) & (parity == 0))
    def _even():
        run(0)

    @pl.when(parity == 1)
    def _odd():
        run(1)


def kernel(x, W, b):
    tokens, d = x.shape
    e = W.shape[0]
    b2 = b.reshape(1, e)
    logits_t = pl.pallas_call(
        _router_block,
        grid=(tokens // _BT,),
        in_specs=[
            pl.BlockSpec(memory_space=pltpu.MemorySpace.HBM),
            pl.BlockSpec((e, d), lambda i: (0, 0)),
            pl.BlockSpec((1, e), lambda i: (0, 0)),
        ],
        out_specs=pl.BlockSpec((e, _BT), lambda i: (0, i)),
        out_shape=jax.ShapeDtypeStruct((e, tokens), jnp.float32),
        scratch_shapes=[
            pltpu.VMEM((_BT, d), jnp.float32),
            pltpu.VMEM((_BT, d), jnp.float32),
            pltpu.SemaphoreType.DMA((2 + _NQ,)),
        ],
    )(x, W, b2)
    return logits_t.T


# speedup vs baseline: 1.0081x; 1.0072x over previous
import jax
import jax.numpy as jnp
from jax.experimental import pallas as pl

_BT = 1024


def _router_block(x_ref, w_ref, b_ref, o_ref):
    xb = x_ref[...].astype(jnp.bfloat16)
    wb = w_ref[...].astype(jnp.bfloat16)
    acc = jax.lax.dot_general(
        wb, xb, (((1,), (1,)), ((), ())),
        preferred_element_type=jnp.float32)
    o_ref[...] = acc + jnp.transpose(b_ref[...], (1, 0))


def kernel(x, W, b):
    tokens, d = x.shape
    e = W.shape[0]
    b2 = b.reshape(1, e)
    logits_t = pl.pallas_call(
        _router_block,
        grid=(tokens // _BT,),
        in_specs=[
            pl.BlockSpec((_BT, d), lambda i: (i, 0)),
            pl.BlockSpec((e, d), lambda i: (0, 0)),
            pl.BlockSpec((1, e), lambda i: (0, 0)),
        ],
        out_specs=pl.BlockSpec((e, _BT), lambda i: (0, i)),
        out_shape=jax.ShapeDtypeStruct((e, tokens), jnp.float32),
    )(x, W, b2)
    return logits_t.T
